# baseline (speedup 1.0000x reference)
import jax
import jax.numpy as jnp
from jax.experimental import pallas as pl
from jax.experimental.pallas import tpu as pltpu

_NUM_LEVELS = 4
_NUM_CODES = 256
_LORA_ALPHA = 1.0
_MU = 0.25
_BLOCK = 512


def _mm_t(a, w):
    return jax.lax.dot_general(
        a.astype(jnp.bfloat16), w.astype(jnp.bfloat16),
        (((1,), (1,)), ((), ())),
        preferred_element_type=jnp.float32)


def _mm_exact(a, w):
    return jax.lax.dot_general(
        a, w, (((1,), (0,)), ((), ())),
        precision=jax.lax.Precision.HIGHEST,
        preferred_element_type=jnp.float32)


def _mlp_fwd(x, Ws, bs):
    h = x
    n = len(Ws)
    for i in range(n):
        h = _mm_t(h, Ws[i][...]) + bs[i][...]
        if i < n - 1:
            h = jnp.maximum(h, 0.0)
    return h


def _rq_block(z, cb_ref, cbbf_ref, c2_ref, idx_ref):
    res = z
    xq = jnp.zeros_like(z)
    cols = []
    total = None
    for l in range(_NUM_LEVELS):
        cb = cb_ref[l]
        c2 = c2_ref[l]
        d = (jnp.sum(res * res, axis=1, keepdims=True)
             - 2.0 * _mm_t(res, cbbf_ref[l]) + c2)
        dmin = jnp.min(d, axis=1, keepdims=True)
        iota = jax.lax.broadcasted_iota(jnp.int32, d.shape, 1)
        idx = jnp.min(jnp.where(d == dmin, iota, _NUM_CODES),
                      axis=1, keepdims=True)
        onehot = (iota == idx).astype(jnp.float32)
        q = _mm_exact(onehot, cb)
        diff = q - res
        s = jnp.sum(diff * diff, keepdims=True)
        total = s if total is None else total + s
        res = res - q
        xq = xq + q
        cols.append(idx)
    idx_ref[...] = jnp.concatenate(cols, axis=1)
    return xq, total


def _body(x_text_ref, x_vis_ref,
          te_w0, te_b0, te_w1, te_b1, te_w2, te_b2, te_w3, te_b3,
          ve_w0, ve_b0, ve_w1, ve_b1, ve_w2, ve_b2, ve_w3, ve_b3,
          tcb_ref, vcb_ref, tcbbf_ref, vcbbf_ref, tc2_ref, vc2_ref,
          tlA_ref, tlB_ref, vlA_ref, vlB_ref,
          td_w0, td_b0, td_w1, td_b1, td_w2, td_b2, td_w3, td_b3,
          vd_w0, vd_b0, vd_w1, vd_b1, vd_w2, vd_b2, vd_w3, vd_b3,
          out_text_ref, out_vis_ref, idx_text_ref, idx_vis_ref,
          losst_ref, lossv_ref):
    z_text = _mlp_fwd(x_text_ref[...],
                      [te_w0, te_w1, te_w2, te_w3],
                      [te_b0, te_b1, te_b2, te_b3])
    z_vis = _mlp_fwd(x_vis_ref[...],
                     [ve_w0, ve_w1, ve_w2, ve_w3],
                     [ve_b0, ve_b1, ve_b2, ve_b3])

    xq_text, sum_text = _rq_block(z_text, tcb_ref, tcbbf_ref, tc2_ref,
                                  idx_text_ref)
    xq_vis, sum_vis = _rq_block(z_vis, vcb_ref, vcbbf_ref, vc2_ref,
                                idx_vis_ref)
    losst_ref[...] = sum_text.reshape(1, 1, 1)
    lossv_ref[...] = sum_vis.reshape(1, 1, 1)

    delta_text = _mm_t(_mm_t(xq_vis, vlB_ref[...]), tlA_ref[...])
    delta_vis = _mm_t(_mm_t(xq_text, tlB_ref[...]), vlA_ref[...])
    xq_text = xq_text + _LORA_ALPHA * delta_text
    xq_vis = xq_vis + _LORA_ALPHA * delta_vis

    out_text_ref[...] = _mlp_fwd(xq_text,
                                 [td_w0, td_w1, td_w2, td_w3],
                                 [td_b0, td_b1, td_b2, td_b3])
    out_vis_ref[...] = _mlp_fwd(xq_vis,
                                [vd_w0, vd_w1, vd_w2, vd_w3],
                                [vd_b0, vd_b1, vd_b2, vd_b3])


def _full_spec(shape):
    nd = len(shape)
    return pl.BlockSpec(shape, lambda i, _nd=nd: (0,) * _nd)


def kernel(x_text, x_vis, params):
    p = params
    B = x_text.shape[0]
    block = _BLOCK if B % _BLOCK == 0 else B
    grid = B // block

    def wb(mlp):
        out = []
        for W, b in zip(mlp['W'], mlp['b']):
            out.append(W.astype(jnp.bfloat16))
            out.append(b.reshape(1, -1))
        return out

    tcb = p['text_codebooks']
    vcb = p['vis_codebooks']
    tc2 = jnp.sum(tcb ** 2, axis=-1)[:, None, :]
    vc2 = jnp.sum(vcb ** 2, axis=-1)[:, None, :]
    operands = ([x_text.astype(jnp.bfloat16), x_vis.astype(jnp.bfloat16)]
                + wb(p['text_enc']) + wb(p['vis_enc'])
                + [tcb, vcb,
                   tcb.astype(jnp.bfloat16), vcb.astype(jnp.bfloat16),
                   tc2, vc2,
                   p['text_lora_A'].astype(jnp.bfloat16),
                   p['text_lora_B'].astype(jnp.bfloat16),
                   p['vis_lora_A'].astype(jnp.bfloat16),
                   p['vis_lora_B'].astype(jnp.bfloat16)]
                + wb(p['text_dec']) + wb(p['vis_dec']))

    d_text = x_text.shape[1]
    d_vis = x_vis.shape[1]

    in_specs = [pl.BlockSpec((block, d_text), lambda i: (i, 0)),
                pl.BlockSpec((block, d_vis), lambda i: (i, 0))]
    for op in operands[2:]:
        in_specs.append(_full_spec(op.shape))

    out_shapes = (
        jax.ShapeDtypeStruct((B, d_text), jnp.float32),
        jax.ShapeDtypeStruct((B, d_vis), jnp.float32),
        jax.ShapeDtypeStruct((B, _NUM_LEVELS), jnp.int32),
        jax.ShapeDtypeStruct((B, _NUM_LEVELS), jnp.int32),
        jax.ShapeDtypeStruct((grid, 1, 1), jnp.float32),
        jax.ShapeDtypeStruct((grid, 1, 1), jnp.float32),
    )
    out_specs = (
        pl.BlockSpec((block, d_text), lambda i: (i, 0)),
        pl.BlockSpec((block, d_vis), lambda i: (i, 0)),
        pl.BlockSpec((block, _NUM_LEVELS), lambda i: (i, 0)),
        pl.BlockSpec((block, _NUM_LEVELS), lambda i: (i, 0)),
        pl.BlockSpec((1, 1, 1), lambda i: (i, 0, 0)),
        pl.BlockSpec((1, 1, 1), lambda i: (i, 0, 0)),
    )

    out_text, out_vis, idx_text, idx_vis, lt, lv = pl.pallas_call(
        _body,
        grid=(grid,),
        in_specs=in_specs,
        out_specs=out_specs,
        out_shape=out_shapes,
        compiler_params=pltpu.CompilerParams(
            dimension_semantics=("parallel",)),
    )(*operands)

    e_dim = p['text_codebooks'].shape[-1]
    scale = (1.0 + _MU) / (_NUM_LEVELS * B * e_dim)
    rq_loss_text = jnp.sum(lt) * scale
    rq_loss_vis = jnp.sum(lv) * scale
    return (out_text, out_vis, rq_loss_text, rq_loss_vis, idx_text, idx_vis)


# speedup vs baseline: 1.0876x; 1.0876x over previous
import jax
import jax.numpy as jnp
from jax.experimental import pallas as pl
from jax.experimental.pallas import tpu as pltpu

_NUM_LEVELS = 4
_NUM_CODES = 256
_LORA_ALPHA = 1.0
_MU = 0.25
_BLOCK = 512


def _mm_t(a, w):
    return jax.lax.dot_general(
        a.astype(jnp.bfloat16), w.astype(jnp.bfloat16),
        (((1,), (1,)), ((), ())),
        preferred_element_type=jnp.float32)


def _mm_exact(a, w):
    return jax.lax.dot_general(
        a, w, (((1,), (0,)), ((), ())),
        precision=jax.lax.Precision.HIGHEST,
        preferred_element_type=jnp.float32)


def _mlp_fwd(x, Ws, bs):
    h = x
    n = len(Ws)
    for i in range(n):
        h = _mm_t(h, Ws[i][...]) + bs[i][...]
        if i < n - 1:
            h = jnp.maximum(h, 0.0)
    return h


def _rq_block(z, cb_ref, cbbf_ref, c2_ref, idx_ref):
    res = z
    xq = jnp.zeros_like(z)
    cols = []
    total = None
    for l in range(_NUM_LEVELS):
        cb = cb_ref[l]
        c2 = c2_ref[l]
        d = (jnp.sum(res * res, axis=1, keepdims=True)
             - 2.0 * _mm_t(res, cbbf_ref[l]) + c2)
        dmin = jnp.min(d, axis=1, keepdims=True)
        iota = jax.lax.broadcasted_iota(jnp.int32, d.shape, 1)
        idx = jnp.min(jnp.where(d == dmin, iota, _NUM_CODES),
                      axis=1, keepdims=True)
        onehot = (iota == idx).astype(jnp.float32)
        q = _mm_exact(onehot, cb)
        diff = q - res
        s = jnp.sum(diff * diff, keepdims=True)
        total = s if total is None else total + s
        res = res - q
        xq = xq + q
        cols.append(idx)
    idx_ref[...] = jnp.concatenate(cols, axis=1)
    return xq, total


def _body(x_text_ref, x_vis_ref,
          te_w0, te_b0, te_w1, te_b1, te_w2, te_b2, te_w3, te_b3,
          ve_w0, ve_b0, ve_w1, ve_b1, ve_w2, ve_b2, ve_w3, ve_b3,
          tcb_ref, vcb_ref, tcbbf_ref, vcbbf_ref, tc2_ref, vc2_ref,
          tlA_ref, tlB_ref, vlA_ref, vlB_ref,
          td_w0, td_b0, td_w1, td_b1, td_w2, td_b2, td_w3, td_b3,
          vd_w0, vd_b0, vd_w1, vd_b1, vd_w2, vd_b2, vd_w3, vd_b3,
          out_text_ref, out_vis_ref, idx_text_ref, idx_vis_ref,
          losst_ref, lossv_ref):
    z_text = _mlp_fwd(x_text_ref[...],
                      [te_w0, te_w1, te_w2, te_w3],
                      [te_b0, te_b1, te_b2, te_b3])
    z_vis = _mlp_fwd(x_vis_ref[...],
                     [ve_w0, ve_w1, ve_w2, ve_w3],
                     [ve_b0, ve_b1, ve_b2, ve_b3])

    xq_text, sum_text = _rq_block(z_text, tcb_ref, tcbbf_ref, tc2_ref,
                                  idx_text_ref)
    xq_vis, sum_vis = _rq_block(z_vis, vcb_ref, vcbbf_ref, vc2_ref,
                                idx_vis_ref)
    losst_ref[...] = sum_text.reshape(1, 1, 1)
    lossv_ref[...] = sum_vis.reshape(1, 1, 1)

    delta_text = _mm_t(_mm_t(xq_vis, vlB_ref[...]), tlA_ref[...])
    delta_vis = _mm_t(_mm_t(xq_text, tlB_ref[...]), vlA_ref[...])
    xq_text = xq_text + _LORA_ALPHA * delta_text
    xq_vis = xq_vis + _LORA_ALPHA * delta_vis

    out_text_ref[...] = _mlp_fwd(xq_text,
                                 [td_w0, td_w1, td_w2, td_w3],
                                 [td_b0, td_b1, td_b2, td_b3])
    out_vis_ref[...] = _mlp_fwd(xq_vis,
                                [vd_w0, vd_w1, vd_w2, vd_w3],
                                [vd_b0, vd_b1, vd_b2, vd_b3])


def _full_spec(shape):
    nd = len(shape)
    return pl.BlockSpec(shape, lambda i, _nd=nd: (0,) * _nd)


def kernel(x_text, x_vis, params):
    p = params
    B = x_text.shape[0]
    block = _BLOCK if B % _BLOCK == 0 else B
    grid = B // block

    def wb(mlp):
        out = []
        for W, b in zip(mlp['W'], mlp['b']):
            out.append(W.astype(jnp.bfloat16))
            out.append(b.reshape(1, -1))
        return out

    tcb = p['text_codebooks']
    vcb = p['vis_codebooks']
    tc2 = jnp.sum(tcb ** 2, axis=-1)[:, None, :]
    vc2 = jnp.sum(vcb ** 2, axis=-1)[:, None, :]
    operands = ([x_text, x_vis]
                + wb(p['text_enc']) + wb(p['vis_enc'])
                + [tcb, vcb,
                   tcb.astype(jnp.bfloat16), vcb.astype(jnp.bfloat16),
                   tc2, vc2,
                   p['text_lora_A'].astype(jnp.bfloat16),
                   p['text_lora_B'].astype(jnp.bfloat16),
                   p['vis_lora_A'].astype(jnp.bfloat16),
                   p['vis_lora_B'].astype(jnp.bfloat16)]
                + wb(p['text_dec']) + wb(p['vis_dec']))

    d_text = x_text.shape[1]
    d_vis = x_vis.shape[1]

    in_specs = [pl.BlockSpec((block, d_text), lambda i: (i, 0)),
                pl.BlockSpec((block, d_vis), lambda i: (i, 0))]
    for op in operands[2:]:
        in_specs.append(_full_spec(op.shape))

    out_shapes = (
        jax.ShapeDtypeStruct((B, d_text), jnp.float32),
        jax.ShapeDtypeStruct((B, d_vis), jnp.float32),
        jax.ShapeDtypeStruct((B, _NUM_LEVELS), jnp.int32),
        jax.ShapeDtypeStruct((B, _NUM_LEVELS), jnp.int32),
        jax.ShapeDtypeStruct((grid, 1, 1), jnp.float32),
        jax.ShapeDtypeStruct((grid, 1, 1), jnp.float32),
    )
    out_specs = (
        pl.BlockSpec((block, d_text), lambda i: (i, 0)),
        pl.BlockSpec((block, d_vis), lambda i: (i, 0)),
        pl.BlockSpec((block, _NUM_LEVELS), lambda i: (i, 0)),
        pl.BlockSpec((block, _NUM_LEVELS), lambda i: (i, 0)),
        pl.BlockSpec((1, 1, 1), lambda i: (i, 0, 0)),
        pl.BlockSpec((1, 1, 1), lambda i: (i, 0, 0)),
    )

    out_text, out_vis, idx_text, idx_vis, lt, lv = pl.pallas_call(
        _body,
        grid=(grid,),
        in_specs=in_specs,
        out_specs=out_specs,
        out_shape=out_shapes,
        compiler_params=pltpu.CompilerParams(
            dimension_semantics=("parallel",)),
    )(*operands)

    e_dim = p['text_codebooks'].shape[-1]
    scale = (1.0 + _MU) / (_NUM_LEVELS * B * e_dim)
    rq_loss_text = jnp.sum(lt) * scale
    rq_loss_vis = jnp.sum(lv) * scale
    return (out_text, out_vis, rq_loss_text, rq_loss_vis, idx_text, idx_vis)


# speedup vs baseline: 1.3297x; 1.2225x over previous
import jax
import jax.numpy as jnp
from jax.experimental import pallas as pl
from jax.experimental.pallas import tpu as pltpu

_NUM_LEVELS = 4
_NUM_CODES = 256
_LORA_ALPHA = 1.0
_MU = 0.25
_BLOCK = 1024


def _mm_t(a, w):
    return jax.lax.dot_general(
        a.astype(jnp.bfloat16), w.astype(jnp.bfloat16),
        (((1,), (1,)), ((), ())),
        preferred_element_type=jnp.float32)


def _mm_exact(a, w):
    return jax.lax.dot_general(
        a, w, (((1,), (0,)), ((), ())),
        precision=jax.lax.Precision.HIGHEST,
        preferred_element_type=jnp.float32)


def _mlp_fwd(x, Ws, bs):
    h = x
    n = len(Ws)
    for i in range(n):
        h = _mm_t(h, Ws[i][...]) + bs[i][...]
        if i < n - 1:
            h = jnp.maximum(h, 0.0)
    return h


def _rq_block(z, cb_ref, cbbf_ref, c2_ref, idx_ref):
    res = z
    xq = jnp.zeros_like(z)
    cols = []
    total = None
    for l in range(_NUM_LEVELS):
        cb = cb_ref[l]
        c2 = c2_ref[l]
        d = (jnp.sum(res * res, axis=1, keepdims=True)
             - 2.0 * _mm_t(res, cbbf_ref[l]) + c2)
        dmin = jnp.min(d, axis=1, keepdims=True)
        iota = jax.lax.broadcasted_iota(jnp.int32, d.shape, 1)
        idx = jnp.min(jnp.where(d == dmin, iota, _NUM_CODES),
                      axis=1, keepdims=True)
        onehot = (iota == idx).astype(jnp.float32)
        q = _mm_exact(onehot, cb)
        diff = q - res
        s = jnp.sum(diff * diff, keepdims=True)
        total = s if total is None else total + s
        res = res - q
        xq = xq + q
        cols.append(idx)
    idx_ref[...] = jnp.concatenate(cols, axis=1)
    return xq, total


def _body(x_text_ref, x_vis_ref,
          te_w0, te_b0, te_w1, te_b1, te_w2, te_b2, te_w3, te_b3,
          ve_w0, ve_b0, ve_w1, ve_b1, ve_w2, ve_b2, ve_w3, ve_b3,
          tcb_ref, vcb_ref, tcbbf_ref, vcbbf_ref, tc2_ref, vc2_ref,
          tlA_ref, tlB_ref, vlA_ref, vlB_ref,
          td_w0, td_b0, td_w1, td_b1, td_w2, td_b2, td_w3, td_b3,
          vd_w0, vd_b0, vd_w1, vd_b1, vd_w2, vd_b2, vd_w3, vd_b3,
          out_text_ref, out_vis_ref, idx_text_ref, idx_vis_ref,
          losst_ref, lossv_ref):
    z_text = _mlp_fwd(x_text_ref[...],
                      [te_w0, te_w1, te_w2, te_w3],
                      [te_b0, te_b1, te_b2, te_b3])
    z_vis = _mlp_fwd(x_vis_ref[...],
                     [ve_w0, ve_w1, ve_w2, ve_w3],
                     [ve_b0, ve_b1, ve_b2, ve_b3])

    xq_text, sum_text = _rq_block(z_text, tcb_ref, tcbbf_ref, tc2_ref,
                                  idx_text_ref)
    xq_vis, sum_vis = _rq_block(z_vis, vcb_ref, vcbbf_ref, vc2_ref,
                                idx_vis_ref)
    losst_ref[...] = sum_text.reshape(1, 1, 1)
    lossv_ref[...] = sum_vis.reshape(1, 1, 1)

    delta_text = _mm_t(_mm_t(xq_vis, vlB_ref[...]), tlA_ref[...])
    delta_vis = _mm_t(_mm_t(xq_text, tlB_ref[...]), vlA_ref[...])
    xq_text = xq_text + _LORA_ALPHA * delta_text
    xq_vis = xq_vis + _LORA_ALPHA * delta_vis

    out_text_ref[...] = _mlp_fwd(xq_text,
                                 [td_w0, td_w1, td_w2, td_w3],
                                 [td_b0, td_b1, td_b2, td_b3])
    out_vis_ref[...] = _mlp_fwd(xq_vis,
                                [vd_w0, vd_w1, vd_w2, vd_w3],
                                [vd_b0, vd_b1, vd_b2, vd_b3])


def _full_spec(shape):
    nd = len(shape)
    return pl.BlockSpec(shape, lambda i, _nd=nd: (0,) * _nd)


def kernel(x_text, x_vis, params):
    p = params
    B = x_text.shape[0]
    block = _BLOCK if B % _BLOCK == 0 else B
    grid = B // block

    def wb(mlp):
        out = []
        for W, b in zip(mlp['W'], mlp['b']):
            out.append(W)
            out.append(b.reshape(1, -1))
        return out

    tcb = p['text_codebooks']
    vcb = p['vis_codebooks']
    tc2 = jnp.sum(tcb ** 2, axis=-1)[:, None, :]
    vc2 = jnp.sum(vcb ** 2, axis=-1)[:, None, :]
    operands = ([x_text, x_vis]
                + wb(p['text_enc']) + wb(p['vis_enc'])
                + [tcb, vcb, tcb, vcb,
                   tc2, vc2,
                   p['text_lora_A'], p['text_lora_B'],
                   p['vis_lora_A'], p['vis_lora_B']]
                + wb(p['text_dec']) + wb(p['vis_dec']))

    d_text = x_text.shape[1]
    d_vis = x_vis.shape[1]

    in_specs = [pl.BlockSpec((block, d_text), lambda i: (i, 0)),
                pl.BlockSpec((block, d_vis), lambda i: (i, 0))]
    for op in operands[2:]:
        in_specs.append(_full_spec(op.shape))

    out_shapes = (
        jax.ShapeDtypeStruct((B, d_text), jnp.float32),
        jax.ShapeDtypeStruct((B, d_vis), jnp.float32),
        jax.ShapeDtypeStruct((B, _NUM_LEVELS), jnp.int32),
        jax.ShapeDtypeStruct((B, _NUM_LEVELS), jnp.int32),
        jax.ShapeDtypeStruct((grid, 1, 1), jnp.float32),
        jax.ShapeDtypeStruct((grid, 1, 1), jnp.float32),
    )
    out_specs = (
        pl.BlockSpec((block, d_text), lambda i: (i, 0)),
        pl.BlockSpec((block, d_vis), lambda i: (i, 0)),
        pl.BlockSpec((block, _NUM_LEVELS), lambda i: (i, 0)),
        pl.BlockSpec((block, _NUM_LEVELS), lambda i: (i, 0)),
        pl.BlockSpec((1, 1, 1), lambda i: (i, 0, 0)),
        pl.BlockSpec((1, 1, 1), lambda i: (i, 0, 0)),
    )

    out_text, out_vis, idx_text, idx_vis, lt, lv = pl.pallas_call(
        _body,
        grid=(grid,),
        in_specs=in_specs,
        out_specs=out_specs,
        out_shape=out_shapes,
        compiler_params=pltpu.CompilerParams(
            dimension_semantics=("parallel",)),
    )(*operands)

    e_dim = p['text_codebooks'].shape[-1]
    scale = (1.0 + _MU) / (_NUM_LEVELS * B * e_dim)
    rq_loss_text = jnp.sum(lt) * scale
    rq_loss_vis = jnp.sum(lv) * scale
    return (out_text, out_vis, rq_loss_text, rq_loss_vis, idx_text, idx_vis)


# speedup vs baseline: 1.8143x; 1.3645x over previous
import jax
import jax.numpy as jnp
from jax.experimental import pallas as pl
from jax.experimental.pallas import tpu as pltpu

_NUM_LEVELS = 4
_NUM_CODES = 256
_LORA_ALPHA = 1.0
_MU = 0.25
_BLOCK = 1024


def _mm_t(a, w):
    return jax.lax.dot_general(
        a.astype(jnp.bfloat16), w.astype(jnp.bfloat16),
        (((1,), (1,)), ((), ())),
        preferred_element_type=jnp.float32)


def _mm_bf(a_bf, w_bf):
    return jax.lax.dot_general(
        a_bf, w_bf, (((1,), (0,)), ((), ())),
        preferred_element_type=jnp.float32)


def _mlp_fwd(x, Ws, bs):
    h = x
    n = len(Ws)
    for i in range(n):
        h = _mm_t(h, Ws[i][...]) + bs[i][...]
        if i < n - 1:
            h = jnp.maximum(h, 0.0)
    return h


def _rq_block(z, cb_ref, c2_ref, idx_ref):
    res = z
    xq = jnp.zeros_like(z)
    cols = []
    total = None
    for l in range(_NUM_LEVELS):
        cb = cb_ref[l]
        hi = cb.astype(jnp.bfloat16)
        r1 = cb - hi.astype(jnp.float32)
        mid = r1.astype(jnp.bfloat16)
        lo = (r1 - mid.astype(jnp.float32)).astype(jnp.bfloat16)
        c2 = c2_ref[l]
        d = (jnp.sum(res * res, axis=1, keepdims=True)
             - 2.0 * _mm_t(res, hi) + c2)
        dmin = jnp.min(d, axis=1, keepdims=True)
        iota = jax.lax.broadcasted_iota(jnp.int32, d.shape, 1)
        idx = jnp.min(jnp.where(d == dmin, iota, _NUM_CODES),
                      axis=1, keepdims=True)
        onehot = (iota == idx).astype(jnp.bfloat16)
        q = ((_mm_bf(onehot, hi) + _mm_bf(onehot, mid))
             + _mm_bf(onehot, lo))
        diff = q - res
        s = jnp.sum(diff * diff, keepdims=True)
        total = s if total is None else total + s
        res = res - q
        xq = xq + q
        cols.append(idx)
    idx_ref[...] = jnp.concatenate(cols, axis=1)
    return xq, total


def _body(x_text_ref, x_vis_ref,
          te_w0, te_b0, te_w1, te_b1, te_w2, te_b2, te_w3, te_b3,
          ve_w0, ve_b0, ve_w1, ve_b1, ve_w2, ve_b2, ve_w3, ve_b3,
          tcb_ref, vcb_ref, tc2_ref, vc2_ref,
          tlA_ref, tlB_ref, vlA_ref, vlB_ref,
          td_w0, td_b0, td_w1, td_b1, td_w2, td_b2, td_w3, td_b3,
          vd_w0, vd_b0, vd_w1, vd_b1, vd_w2, vd_b2, vd_w3, vd_b3,
          out_text_ref, out_vis_ref, idx_text_ref, idx_vis_ref,
          losst_ref, lossv_ref):
    z_text = _mlp_fwd(x_text_ref[...],
                      [te_w0, te_w1, te_w2, te_w3],
                      [te_b0, te_b1, te_b2, te_b3])
    z_vis = _mlp_fwd(x_vis_ref[...],
                     [ve_w0, ve_w1, ve_w2, ve_w3],
                     [ve_b0, ve_b1, ve_b2, ve_b3])

    xq_text, sum_text = _rq_block(z_text, tcb_ref, tc2_ref, idx_text_ref)
    xq_vis, sum_vis = _rq_block(z_vis, vcb_ref, vc2_ref, idx_vis_ref)
    losst_ref[...] = sum_text.reshape(1, 1, 1)
    lossv_ref[...] = sum_vis.reshape(1, 1, 1)

    delta_text = _mm_t(_mm_t(xq_vis, vlB_ref[...]), tlA_ref[...])
    delta_vis = _mm_t(_mm_t(xq_text, tlB_ref[...]), vlA_ref[...])
    xq_text = xq_text + _LORA_ALPHA * delta_text
    xq_vis = xq_vis + _LORA_ALPHA * delta_vis

    out_text_ref[...] = _mlp_fwd(xq_text,
                                 [td_w0, td_w1, td_w2, td_w3],
                                 [td_b0, td_b1, td_b2, td_b3])
    out_vis_ref[...] = _mlp_fwd(xq_vis,
                                [vd_w0, vd_w1, vd_w2, vd_w3],
                                [vd_b0, vd_b1, vd_b2, vd_b3])


def _full_spec(shape):
    nd = len(shape)
    return pl.BlockSpec(shape, lambda i, _nd=nd: (0,) * _nd)


def kernel(x_text, x_vis, params):
    p = params
    B = x_text.shape[0]
    block = _BLOCK if B % _BLOCK == 0 else B
    grid = B // block

    def wb(mlp):
        out = []
        for W, b in zip(mlp['W'], mlp['b']):
            out.append(W)
            out.append(b.reshape(1, -1))
        return out

    tcb = p['text_codebooks']
    vcb = p['vis_codebooks']
    tc2 = jnp.sum(tcb ** 2, axis=-1)[:, None, :]
    vc2 = jnp.sum(vcb ** 2, axis=-1)[:, None, :]

    operands = ([x_text, x_vis]
                + wb(p['text_enc']) + wb(p['vis_enc'])
                + [tcb, vcb,
                   tc2, vc2,
                   p['text_lora_A'], p['text_lora_B'],
                   p['vis_lora_A'], p['vis_lora_B']]
                + wb(p['text_dec']) + wb(p['vis_dec']))

    d_text = x_text.shape[1]
    d_vis = x_vis.shape[1]

    in_specs = [pl.BlockSpec((block, d_text), lambda i: (i, 0)),
                pl.BlockSpec((block, d_vis), lambda i: (i, 0))]
    for op in operands[2:]:
        in_specs.append(_full_spec(op.shape))

    out_shapes = (
        jax.ShapeDtypeStruct((B, d_text), jnp.float32),
        jax.ShapeDtypeStruct((B, d_vis), jnp.float32),
        jax.ShapeDtypeStruct((B, _NUM_LEVELS), jnp.int32),
        jax.ShapeDtypeStruct((B, _NUM_LEVELS), jnp.int32),
        jax.ShapeDtypeStruct((grid, 1, 1), jnp.float32),
        jax.ShapeDtypeStruct((grid, 1, 1), jnp.float32),
    )
    out_specs = (
        pl.BlockSpec((block, d_text), lambda i: (i, 0)),
        pl.BlockSpec((block, d_vis), lambda i: (i, 0)),
        pl.BlockSpec((block, _NUM_LEVELS), lambda i: (i, 0)),
        pl.BlockSpec((block, _NUM_LEVELS), lambda i: (i, 0)),
        pl.BlockSpec((1, 1, 1), lambda i: (i, 0, 0)),
        pl.BlockSpec((1, 1, 1), lambda i: (i, 0, 0)),
    )

    out_text, out_vis, idx_text, idx_vis, lt, lv = pl.pallas_call(
        _body,
        grid=(grid,),
        in_specs=in_specs,
        out_specs=out_specs,
        out_shape=out_shapes,
        compiler_params=pltpu.CompilerParams(
            dimension_semantics=("parallel",)),
    )(*operands)

    e_dim = p['text_codebooks'].shape[-1]
    scale = (1.0 + _MU) / (_NUM_LEVELS * B * e_dim)
    rq_loss_text = jnp.sum(lt) * scale
    rq_loss_vis = jnp.sum(lv) * scale
    return (out_text, out_vis, rq_loss_text, rq_loss_vis, idx_text, idx_vis)


# speedup vs baseline: 1.8284x; 1.0078x over previous
import jax
import jax.numpy as jnp
from jax.experimental import pallas as pl
from jax.experimental.pallas import tpu as pltpu

_NUM_LEVELS = 4
_NUM_CODES = 256
_LORA_ALPHA = 1.0
_MU = 0.25
_BLOCK = 1024


def _mm_t(a, w):
    return jax.lax.dot_general(
        a.astype(jnp.bfloat16), w.astype(jnp.bfloat16),
        (((1,), (1,)), ((), ())),
        preferred_element_type=jnp.float32)


def _mm_bf(a_bf, w_bf):
    return jax.lax.dot_general(
        a_bf, w_bf, (((1,), (0,)), ((), ())),
        preferred_element_type=jnp.float32)


def _mlp_fwd(x, Ws, bs):
    h = x
    n = len(Ws)
    for i in range(n):
        h = _mm_t(h, Ws[i][...]) + bs[i][...]
        if i < n - 1:
            h = jnp.maximum(h, 0.0)
    return h


def _rq_block(z, cb_ref, c2_ref, idx_ref):
    res = z
    xq = jnp.zeros_like(z)
    cols = []
    total = None
    for l in range(_NUM_LEVELS):
        cb = cb_ref[l]
        hi = cb.astype(jnp.bfloat16)
        r1 = cb - hi.astype(jnp.float32)
        mid = r1.astype(jnp.bfloat16)
        lo = (r1 - mid.astype(jnp.float32)).astype(jnp.bfloat16)
        c2 = c2_ref[l]
        d = (jnp.sum(res * res, axis=1, keepdims=True)
             - _mm_t(res, hi + hi) + c2)
        dmin = jnp.min(d, axis=1, keepdims=True)
        iota = jax.lax.broadcasted_iota(jnp.int32, d.shape, 1)
        idx = jnp.min(jnp.where(d == dmin, iota, _NUM_CODES),
                      axis=1, keepdims=True)
        onehot = (iota == idx).astype(jnp.bfloat16)
        q = ((_mm_bf(onehot, hi) + _mm_bf(onehot, mid))
             + _mm_bf(onehot, lo))
        diff = q - res
        s = jnp.sum(diff * diff, keepdims=True)
        total = s if total is None else total + s
        res = res - q
        xq = xq + q
        cols.append(idx)
    idx_ref[...] = jnp.concatenate(cols, axis=1)
    return xq, total


def _body(x_text_ref, x_vis_ref,
          te_w0, te_b0, te_w1, te_b1, te_w2, te_b2, te_w3, te_b3,
          ve_w0, ve_b0, ve_w1, ve_b1, ve_w2, ve_b2, ve_w3, ve_b3,
          tcb_ref, vcb_ref, tc2_ref, vc2_ref,
          tlA_ref, tlB_ref, vlA_ref, vlB_ref,
          td_w0, td_b0, td_w1, td_b1, td_w2, td_b2, td_w3, td_b3,
          vd_w0, vd_b0, vd_w1, vd_b1, vd_w2, vd_b2, vd_w3, vd_b3,
          out_text_ref, out_vis_ref, idx_text_ref, idx_vis_ref,
          losst_ref, lossv_ref):
    z_text = _mlp_fwd(x_text_ref[...],
                      [te_w0, te_w1, te_w2, te_w3],
                      [te_b0, te_b1, te_b2, te_b3])
    z_vis = _mlp_fwd(x_vis_ref[...],
                     [ve_w0, ve_w1, ve_w2, ve_w3],
                     [ve_b0, ve_b1, ve_b2, ve_b3])

    xq_text, sum_text = _rq_block(z_text, tcb_ref, tc2_ref, idx_text_ref)
    xq_vis, sum_vis = _rq_block(z_vis, vcb_ref, vc2_ref, idx_vis_ref)
    losst_ref[...] = sum_text.reshape(1, 1, 1)
    lossv_ref[...] = sum_vis.reshape(1, 1, 1)

    delta_text = _mm_t(_mm_t(xq_vis, vlB_ref[...]), tlA_ref[...])
    delta_vis = _mm_t(_mm_t(xq_text, tlB_ref[...]), vlA_ref[...])
    xq_text = xq_text + _LORA_ALPHA * delta_text
    xq_vis = xq_vis + _LORA_ALPHA * delta_vis

    out_text_ref[...] = _mlp_fwd(xq_text,
                                 [td_w0, td_w1, td_w2, td_w3],
                                 [td_b0, td_b1, td_b2, td_b3])
    out_vis_ref[...] = _mlp_fwd(xq_vis,
                                [vd_w0, vd_w1, vd_w2, vd_w3],
                                [vd_b0, vd_b1, vd_b2, vd_b3])


def _full_spec(shape):
    nd = len(shape)
    return pl.BlockSpec(shape, lambda i, _nd=nd: (0,) * _nd)


def kernel(x_text, x_vis, params):
    p = params
    B = x_text.shape[0]
    block = _BLOCK if B % _BLOCK == 0 else B
    grid = B // block

    def wb(mlp):
        out = []
        for W, b in zip(mlp['W'], mlp['b']):
            out.append(W)
            out.append(b.reshape(1, -1))
        return out

    tcb = p['text_codebooks']
    vcb = p['vis_codebooks']
    tc2 = jnp.sum(tcb ** 2, axis=-1)[:, None, :]
    vc2 = jnp.sum(vcb ** 2, axis=-1)[:, None, :]

    operands = ([x_text, x_vis]
                + wb(p['text_enc']) + wb(p['vis_enc'])
                + [tcb, vcb,
                   tc2, vc2,
                   p['text_lora_A'], p['text_lora_B'],
                   p['vis_lora_A'], p['vis_lora_B']]
                + wb(p['text_dec']) + wb(p['vis_dec']))

    d_text = x_text.shape[1]
    d_vis = x_vis.shape[1]

    in_specs = [pl.BlockSpec((block, d_text), lambda i: (i, 0)),
                pl.BlockSpec((block, d_vis), lambda i: (i, 0))]
    for op in operands[2:]:
        in_specs.append(_full_spec(op.shape))

    out_shapes = (
        jax.ShapeDtypeStruct((B, d_text), jnp.float32),
        jax.ShapeDtypeStruct((B, d_vis), jnp.float32),
        jax.ShapeDtypeStruct((B, _NUM_LEVELS), jnp.int32),
        jax.ShapeDtypeStruct((B, _NUM_LEVELS), jnp.int32),
        jax.ShapeDtypeStruct((grid, 1, 1), jnp.float32),
        jax.ShapeDtypeStruct((grid, 1, 1), jnp.float32),
    )
    out_specs = (
        pl.BlockSpec((block, d_text), lambda i: (i, 0)),
        pl.BlockSpec((block, d_vis), lambda i: (i, 0)),
        pl.BlockSpec((block, _NUM_LEVELS), lambda i: (i, 0)),
        pl.BlockSpec((block, _NUM_LEVELS), lambda i: (i, 0)),
        pl.BlockSpec((1, 1, 1), lambda i: (i, 0, 0)),
        pl.BlockSpec((1, 1, 1), lambda i: (i, 0, 0)),
    )

    out_text, out_vis, idx_text, idx_vis, lt, lv = pl.pallas_call(
        _body,
        grid=(grid,),
        in_specs=in_specs,
        out_specs=out_specs,
        out_shape=out_shapes,
        compiler_params=pltpu.CompilerParams(
            dimension_semantics=("parallel",)),
    )(*operands)

    e_dim = p['text_codebooks'].shape[-1]
    scale = (1.0 + _MU) / (_NUM_LEVELS * B * e_dim)
    rq_loss_text = jnp.sum(lt) * scale
    rq_loss_vis = jnp.sum(lv) * scale
    return (out_text, out_vis, rq_loss_text, rq_loss_vis, idx_text, idx_vis)


# speedup vs baseline: 1.8421x; 1.0075x over previous
import jax
import jax.numpy as jnp
from jax.experimental import pallas as pl
from jax.experimental.pallas import tpu as pltpu

_NUM_LEVELS = 4
_NUM_CODES = 256
_LORA_ALPHA = 1.0
_MU = 0.25
_BLOCK = 1024


def _mm_t(a, w):
    return jax.lax.dot_general(
        a.astype(jnp.bfloat16), w.astype(jnp.bfloat16),
        (((1,), (1,)), ((), ())),
        preferred_element_type=jnp.float32)


def _mm_bf(a_bf, w_bf):
    return jax.lax.dot_general(
        a_bf, w_bf, (((1,), (0,)), ((), ())),
        preferred_element_type=jnp.float32)


def _mlp_fwd(x, Ws, bs):
    h = x
    n = len(Ws)
    for i in range(n):
        h = _mm_t(h, Ws[i][...]) + bs[i][...]
        if i < n - 1:
            h = jnp.maximum(h, 0.0)
    return h


def _rq_block(z, cb_ref, c2_ref, idx_ref):
    res = z
    xq = jnp.zeros_like(z)
    cols = []
    total = None
    for l in range(_NUM_LEVELS):
        cb = cb_ref[l]
        hi = cb.astype(jnp.bfloat16)
        r1 = cb - hi.astype(jnp.float32)
        mid = r1.astype(jnp.bfloat16)
        lo = (r1 - mid.astype(jnp.float32)).astype(jnp.bfloat16)
        c2 = c2_ref[l]
        d = (jnp.sum(res * res, axis=1, keepdims=True)
             - _mm_t(res, hi + hi) + c2)
        dmin = jnp.min(d, axis=1, keepdims=True)
        iota = jax.lax.broadcasted_iota(jnp.int32, d.shape, 1)
        idx = jnp.min(jnp.where(d == dmin, iota, _NUM_CODES),
                      axis=1, keepdims=True)
        onehot = (iota == idx).astype(jnp.bfloat16)
        q = ((_mm_bf(onehot, hi) + _mm_bf(onehot, mid))
             + _mm_bf(onehot, lo))
        res = res - q
        s = jnp.sum(res * res, keepdims=True)
        total = s if total is None else total + s
        xq = xq + q
        cols.append(idx)
    idx_ref[...] = jnp.concatenate(cols, axis=1)
    return xq, total


def _body(x_text_ref, x_vis_ref,
          te_w0, te_b0, te_w1, te_b1, te_w2, te_b2, te_w3, te_b3,
          ve_w0, ve_b0, ve_w1, ve_b1, ve_w2, ve_b2, ve_w3, ve_b3,
          tcb_ref, vcb_ref, tc2_ref, vc2_ref,
          tlA_ref, tlB_ref, vlA_ref, vlB_ref,
          td_w0, td_b0, td_w1, td_b1, td_w2, td_b2, td_w3, td_b3,
          vd_w0, vd_b0, vd_w1, vd_b1, vd_w2, vd_b2, vd_w3, vd_b3,
          out_text_ref, out_vis_ref, idx_text_ref, idx_vis_ref,
          losst_ref, lossv_ref):
    z_text = _mlp_fwd(x_text_ref[...],
                      [te_w0, te_w1, te_w2, te_w3],
                      [te_b0, te_b1, te_b2, te_b3])
    z_vis = _mlp_fwd(x_vis_ref[...],
                     [ve_w0, ve_w1, ve_w2, ve_w3],
                     [ve_b0, ve_b1, ve_b2, ve_b3])

    xq_text, sum_text = _rq_block(z_text, tcb_ref, tc2_ref, idx_text_ref)
    xq_vis, sum_vis = _rq_block(z_vis, vcb_ref, vc2_ref, idx_vis_ref)
    losst_ref[...] = sum_text.reshape(1, 1, 1)
    lossv_ref[...] = sum_vis.reshape(1, 1, 1)

    delta_text = _mm_t(_mm_t(xq_vis, vlB_ref[...]), tlA_ref[...])
    delta_vis = _mm_t(_mm_t(xq_text, tlB_ref[...]), vlA_ref[...])
    xq_text = xq_text + _LORA_ALPHA * delta_text
    xq_vis = xq_vis + _LORA_ALPHA * delta_vis

    out_text_ref[...] = _mlp_fwd(xq_text,
                                 [td_w0, td_w1, td_w2, td_w3],
                                 [td_b0, td_b1, td_b2, td_b3])
    out_vis_ref[...] = _mlp_fwd(xq_vis,
                                [vd_w0, vd_w1, vd_w2, vd_w3],
                                [vd_b0, vd_b1, vd_b2, vd_b3])


def _full_spec(shape):
    nd = len(shape)
    return pl.BlockSpec(shape, lambda i, _nd=nd: (0,) * _nd)


def kernel(x_text, x_vis, params):
    p = params
    B = x_text.shape[0]
    block = _BLOCK if B % _BLOCK == 0 else B
    grid = B // block

    def wb(mlp):
        out = []
        for W, b in zip(mlp['W'], mlp['b']):
            out.append(W)
            out.append(b.reshape(1, -1))
        return out

    tcb = p['text_codebooks']
    vcb = p['vis_codebooks']
    tc2 = jnp.sum(tcb ** 2, axis=-1)[:, None, :]
    vc2 = jnp.sum(vcb ** 2, axis=-1)[:, None, :]

    operands = ([x_text, x_vis]
                + wb(p['text_enc']) + wb(p['vis_enc'])
                + [tcb, vcb,
                   tc2, vc2,
                   p['text_lora_A'], p['text_lora_B'],
                   p['vis_lora_A'], p['vis_lora_B']]
                + wb(p['text_dec']) + wb(p['vis_dec']))

    d_text = x_text.shape[1]
    d_vis = x_vis.shape[1]

    in_specs = [pl.BlockSpec((block, d_text), lambda i: (i, 0)),
                pl.BlockSpec((block, d_vis), lambda i: (i, 0))]
    for op in operands[2:]:
        in_specs.append(_full_spec(op.shape))

    out_shapes = (
        jax.ShapeDtypeStruct((B, d_text), jnp.float32),
        jax.ShapeDtypeStruct((B, d_vis), jnp.float32),
        jax.ShapeDtypeStruct((B, _NUM_LEVELS), jnp.int32),
        jax.ShapeDtypeStruct((B, _NUM_LEVELS), jnp.int32),
        jax.ShapeDtypeStruct((grid, 1, 1), jnp.float32),
        jax.ShapeDtypeStruct((grid, 1, 1), jnp.float32),
    )
    out_specs = (
        pl.BlockSpec((block, d_text), lambda i: (i, 0)),
        pl.BlockSpec((block, d_vis), lambda i: (i, 0)),
        pl.BlockSpec((block, _NUM_LEVELS), lambda i: (i, 0)),
        pl.BlockSpec((block, _NUM_LEVELS), lambda i: (i, 0)),
        pl.BlockSpec((1, 1, 1), lambda i: (i, 0, 0)),
        pl.BlockSpec((1, 1, 1), lambda i: (i, 0, 0)),
    )

    out_text, out_vis, idx_text, idx_vis, lt, lv = pl.pallas_call(
        _body,
        grid=(grid,),
        in_specs=in_specs,
        out_specs=out_specs,
        out_shape=out_shapes,
        compiler_params=pltpu.CompilerParams(
            dimension_semantics=("parallel",)),
    )(*operands)

    e_dim = p['text_codebooks'].shape[-1]
    scale = (1.0 + _MU) / (_NUM_LEVELS * B * e_dim)
    rq_loss_text = jnp.sum(lt) * scale
    rq_loss_vis = jnp.sum(lv) * scale
    return (out_text, out_vis, rq_loss_text, rq_loss_vis, idx_text, idx_vis)
